# baseline (speedup 1.0000x reference)
import functools

import jax
import jax.numpy as jnp
from jax import lax
from jax.experimental import pallas as pl
from jax.experimental.pallas import tpu as pltpu
from jax.experimental.pallas import tpu_sc as plsc

N_TOKENS = 32768
D_MODEL = 768
N_EXPERTS = 64
CAPACITY = 512
GATE_H = 64
EXP_H = 128
FLAT_ROWS = N_EXPERTS * CAPACITY + 1

TB = 1024
N_TBLOCKS = N_TOKENS // TB

NC, NS = 2, 16
NW = NC * NS
TOK_PER_W = N_TOKENS // NW
K_CHUNK = 64
N_CHUNKS = TOK_PER_W // K_CHUNK


def _gate_body(x_ref, wg1_ref, bg1_ref, wg2_ref, bg2_ref, slot_ref, counts_ref):
    i = pl.program_id(0)

    @pl.when(i == 0)
    def _():
        counts_ref[...] = jnp.zeros_like(counts_ref)

    x = x_ref[...]
    gh = jnp.dot(x, wg1_ref[...], preferred_element_type=jnp.float32)
    gh = jnp.maximum(gh + bg1_ref[...], 0.0)
    logits = jnp.dot(gh, wg2_ref[...], preferred_element_type=jnp.float32)
    logits = logits + bg2_ref[...]

    m = jnp.max(logits, axis=1, keepdims=True)
    iota_e = lax.broadcasted_iota(jnp.int32, (TB, N_EXPERTS), 1)
    sel = logits == m
    expert = jnp.min(jnp.where(sel, iota_e, N_EXPERTS), axis=1)
    first = expert[:, None] == iota_e

    ii = lax.broadcasted_iota(jnp.int32, (TB, TB), 0)
    jj = lax.broadcasted_iota(jnp.int32, (TB, TB), 1)
    tri = (jj <= ii).astype(jnp.bfloat16)
    onehot = first.astype(jnp.bfloat16)
    pos_incl = jnp.dot(tri, onehot, preferred_element_type=jnp.float32)

    prev = counts_ref[...]
    pos_all = pos_incl + prev
    pos_tok = jnp.sum(jnp.where(first, pos_all, 0.0), axis=1) - 1.0
    keep = pos_tok < CAPACITY
    slot = jnp.where(
        keep,
        expert * CAPACITY + pos_tok.astype(jnp.int32),
        N_EXPERTS * CAPACITY,
    )
    slot_ref[...] = slot[None, None, :]
    counts_ref[...] = prev + jnp.sum(first.astype(jnp.float32), axis=0,
                                     keepdims=True)


def _gate_route(x, wg1, bg1, wg2, bg2):
    return pl.pallas_call(
        _gate_body,
        grid=(N_TBLOCKS,),
        in_specs=[
            pl.BlockSpec((TB, D_MODEL), lambda i: (i, 0)),
            pl.BlockSpec((D_MODEL, GATE_H), lambda i: (0, 0)),
            pl.BlockSpec((1, GATE_H), lambda i: (0, 0)),
            pl.BlockSpec((GATE_H, N_EXPERTS), lambda i: (0, 0)),
            pl.BlockSpec((1, N_EXPERTS), lambda i: (0, 0)),
        ],
        out_specs=pl.BlockSpec((1, 1, TB), lambda i: (i, 0, 0)),
        out_shape=jax.ShapeDtypeStruct((N_TBLOCKS, 1, TB), jnp.int32),
        scratch_shapes=[pltpu.VMEM((1, N_EXPERTS), jnp.float32)],
    )(x, wg1, bg1, wg2, bg2)


def _expert_body(disp_ref, w1_ref, b1_ref, w2_ref, b2_ref, out_ref):
    e = pl.program_id(0)

    @pl.when(e < N_EXPERTS)
    def _():
        xb = disp_ref[...].astype(jnp.bfloat16)
        h = jnp.dot(xb, w1_ref[0].astype(jnp.bfloat16),
                    preferred_element_type=jnp.float32)
        h = jnp.maximum(h + b1_ref[0], 0.0)
        z = jnp.dot(h.astype(jnp.bfloat16), w2_ref[0].astype(jnp.bfloat16),
                    preferred_element_type=jnp.float32)
        z = z + b2_ref[0]
        out_ref[...] = jax.nn.softmax(z, axis=-1)

    @pl.when(e == N_EXPERTS)
    def _():
        out_ref[...] = jnp.zeros_like(out_ref)


def _experts(disp, w1, b1, w2, b2):
    clamp = lambda e: jnp.minimum(e, N_EXPERTS - 1)
    return pl.pallas_call(
        _expert_body,
        grid=(N_EXPERTS + 1,),
        in_specs=[
            pl.BlockSpec((CAPACITY, D_MODEL), lambda e: (clamp(e), 0)),
            pl.BlockSpec((1, D_MODEL, EXP_H), lambda e: (clamp(e), 0, 0)),
            pl.BlockSpec((1, 1, EXP_H), lambda e: (clamp(e), 0, 0)),
            pl.BlockSpec((1, EXP_H, D_MODEL), lambda e: (clamp(e), 0, 0)),
            pl.BlockSpec((1, 1, D_MODEL), lambda e: (clamp(e), 0, 0)),
        ],
        out_specs=pl.BlockSpec((CAPACITY, D_MODEL), lambda e: (e, 0)),
        out_shape=jax.ShapeDtypeStruct((FLAT_ROWS, D_MODEL), jnp.float32),
    )(disp, w1, b1.reshape(N_EXPERTS, 1, EXP_H), w2,
      b2.reshape(N_EXPERTS, 1, D_MODEL))


def _worker_id():
    return lax.axis_index("s") * NC + lax.axis_index("c")


@functools.cache
def _sc_kernels():
    mesh = plsc.VectorSubcoreMesh(core_axis_name="c", subcore_axis_name="s",
                                  num_cores=NC, num_subcores=NS)
    scratch = [
        pltpu.VMEM((N_CHUNKS, K_CHUNK), jnp.int32),
        pltpu.VMEM((K_CHUNK, D_MODEL), jnp.float32),
        pltpu.VMEM((K_CHUNK, D_MODEL), jnp.float32),
        pltpu.SemaphoreType.DMA,
        pltpu.SemaphoreType.DMA,
    ]

    @functools.partial(
        pl.kernel,
        mesh=mesh,
        out_type=jax.ShapeDtypeStruct((FLAT_ROWS, D_MODEL), jnp.float32),
        scratch_types=scratch,
    )
    def dispatch(x_hbm, slot_hbm, out_hbm, idx_v, rows0, rows1, sem0, sem1):
        wid = _worker_id()
        rows, sems = (rows0, rows1), (sem0, sem1)
        pltpu.sync_copy(slot_hbm.at[wid], idx_v)
        base = wid * TOK_PER_W
        pltpu.sync_copy(x_hbm.at[pl.ds(base, K_CHUNK)], rows0)
        handles = [None, None]
        for j in range(N_CHUNKS):
            b = j % 2
            handles[b] = pltpu.async_copy(rows[b], out_hbm.at[idx_v.at[j]],
                                          sems[b])
            if j + 1 < N_CHUNKS:
                nb = (j + 1) % 2
                if handles[nb] is not None:
                    handles[nb].wait()
                pltpu.sync_copy(
                    x_hbm.at[pl.ds(base + (j + 1) * K_CHUNK, K_CHUNK)],
                    rows[nb])
        handles[0].wait()
        handles[1].wait()

    @functools.partial(
        pl.kernel,
        mesh=mesh,
        out_type=jax.ShapeDtypeStruct((N_TOKENS, D_MODEL), jnp.float32),
        scratch_types=scratch,
    )
    def combine(flat_hbm, slot_hbm, out_hbm, idx_v, rows0, rows1, sem0, sem1):
        wid = _worker_id()
        rows, sems = (rows0, rows1), (sem0, sem1)
        pltpu.sync_copy(slot_hbm.at[wid], idx_v)
        base = wid * TOK_PER_W
        handles = [
            pltpu.async_copy(flat_hbm.at[idx_v.at[0]], rows0, sem0), None]
        for j in range(N_CHUNKS):
            b = j % 2
            if j + 1 < N_CHUNKS:
                handles[1 - b] = pltpu.async_copy(
                    flat_hbm.at[idx_v.at[j + 1]], rows[1 - b], sems[1 - b])
            handles[b].wait()
            pltpu.sync_copy(rows[b],
                            out_hbm.at[pl.ds(base + j * K_CHUNK, K_CHUNK)])

    return dispatch, combine


def kernel(inputs, Wg1, bg1, Wg2, bg2, W1, b1, W2, b2):
    slot = _gate_route(inputs, Wg1, bg1.reshape(1, -1), Wg2, bg2.reshape(1, -1))
    slot3 = slot.reshape(NW, N_CHUNKS, K_CHUNK)
    dispatch, combine = _sc_kernels()
    disp = dispatch(inputs, slot3)
    flat = _experts(disp, W1, b1, W2, b2)
    return combine(flat, slot3)


# speedup vs baseline: 1.0718x; 1.0718x over previous
import functools

import jax
import jax.numpy as jnp
from jax import lax
from jax.experimental import pallas as pl
from jax.experimental.pallas import tpu as pltpu
from jax.experimental.pallas import tpu_sc as plsc

N_TOKENS = 32768
D_MODEL = 768
N_EXPERTS = 64
CAPACITY = 512
GATE_H = 64
EXP_H = 128
FLAT_ROWS = N_EXPERTS * CAPACITY + 1

TB = 1024
N_TBLOCKS = N_TOKENS // TB
D_HALF = D_MODEL // 2


def _pack_bf16(x32):
    lo = lax.bitcast_convert_type(x32[:, :D_HALF].astype(jnp.bfloat16),
                                  jnp.uint16).astype(jnp.uint32)
    hi = lax.bitcast_convert_type(x32[:, D_HALF:].astype(jnp.bfloat16),
                                  jnp.uint16).astype(jnp.uint32)
    return lax.bitcast_convert_type(lo | (hi << 16), jnp.float32)


def _unpack_bf16(p32):
    u = lax.bitcast_convert_type(p32, jnp.uint32)
    lo = lax.bitcast_convert_type((u & 0xFFFF).astype(jnp.uint16),
                                  jnp.bfloat16)
    hi = lax.bitcast_convert_type((u >> 16).astype(jnp.uint16), jnp.bfloat16)
    return jnp.concatenate([lo, hi], axis=1)

NC, NS = 2, 16
NW = NC * NS
TOK_PER_W = N_TOKENS // NW
K_CHUNK = 128
N_CHUNKS = TOK_PER_W // K_CHUNK


def _gate_body(x_ref, wg1_ref, bg1_ref, wg2_ref, bg2_ref, slot_ref, xbf_ref,
               counts_ref):
    i = pl.program_id(0)

    @pl.when(i == 0)
    def _():
        counts_ref[...] = jnp.zeros_like(counts_ref)

    x = x_ref[...]
    xbf_ref[...] = _pack_bf16(x)
    gh = jnp.dot(x, wg1_ref[...], preferred_element_type=jnp.float32)
    gh = jnp.maximum(gh + bg1_ref[...], 0.0)
    logits = jnp.dot(gh, wg2_ref[...], preferred_element_type=jnp.float32)
    logits = logits + bg2_ref[...]

    m = jnp.max(logits, axis=1, keepdims=True)
    iota_e = lax.broadcasted_iota(jnp.int32, (TB, N_EXPERTS), 1)
    sel = logits == m
    expert = jnp.min(jnp.where(sel, iota_e, N_EXPERTS), axis=1)
    first = expert[:, None] == iota_e

    ii = lax.broadcasted_iota(jnp.int32, (TB, TB), 0)
    jj = lax.broadcasted_iota(jnp.int32, (TB, TB), 1)
    tri = (jj <= ii).astype(jnp.bfloat16)
    onehot = first.astype(jnp.bfloat16)
    pos_incl = jnp.dot(tri, onehot, preferred_element_type=jnp.float32)

    prev = counts_ref[...]
    pos_all = pos_incl + prev
    pos_tok = jnp.sum(jnp.where(first, pos_all, 0.0), axis=1) - 1.0
    keep = pos_tok < CAPACITY
    slot = jnp.where(
        keep,
        expert * CAPACITY + pos_tok.astype(jnp.int32),
        N_EXPERTS * CAPACITY,
    )
    slot_ref[...] = slot[None, None, :]
    counts_ref[...] = prev + jnp.sum(first.astype(jnp.float32), axis=0,
                                     keepdims=True)


def _gate_route(x, wg1, bg1, wg2, bg2):
    return pl.pallas_call(
        _gate_body,
        grid=(N_TBLOCKS,),
        in_specs=[
            pl.BlockSpec((TB, D_MODEL), lambda i: (i, 0)),
            pl.BlockSpec((D_MODEL, GATE_H), lambda i: (0, 0)),
            pl.BlockSpec((1, GATE_H), lambda i: (0, 0)),
            pl.BlockSpec((GATE_H, N_EXPERTS), lambda i: (0, 0)),
            pl.BlockSpec((1, N_EXPERTS), lambda i: (0, 0)),
        ],
        out_specs=[
            pl.BlockSpec((1, 1, TB), lambda i: (i, 0, 0)),
            pl.BlockSpec((TB, D_HALF), lambda i: (i, 0)),
        ],
        out_shape=[
            jax.ShapeDtypeStruct((N_TBLOCKS, 1, TB), jnp.int32),
            jax.ShapeDtypeStruct((N_TOKENS, D_HALF), jnp.float32),
        ],
        scratch_shapes=[pltpu.VMEM((1, N_EXPERTS), jnp.float32)],
    )(x, wg1, bg1, wg2, bg2)


def _expert_body(disp_ref, w1_ref, b1_ref, w2_ref, b2_ref, out_ref):
    e = pl.program_id(0)

    @pl.when(e < N_EXPERTS)
    def _():
        xb = _unpack_bf16(disp_ref[...])
        h = jnp.dot(xb, w1_ref[0].astype(jnp.bfloat16),
                    preferred_element_type=jnp.float32)
        h = jnp.maximum(h + b1_ref[0], 0.0)
        z = jnp.dot(h.astype(jnp.bfloat16), w2_ref[0].astype(jnp.bfloat16),
                    preferred_element_type=jnp.float32)
        z = z + b2_ref[0]
        out_ref[...] = _pack_bf16(jax.nn.softmax(z, axis=-1))

    @pl.when(e == N_EXPERTS)
    def _():
        out_ref[...] = jnp.zeros_like(out_ref)


def _experts(disp, w1, b1, w2, b2):
    clamp = lambda e: jnp.minimum(e, N_EXPERTS - 1)
    return pl.pallas_call(
        _expert_body,
        grid=(N_EXPERTS + 1,),
        in_specs=[
            pl.BlockSpec((CAPACITY, D_HALF), lambda e: (clamp(e), 0)),
            pl.BlockSpec((1, D_MODEL, EXP_H), lambda e: (clamp(e), 0, 0)),
            pl.BlockSpec((1, 1, EXP_H), lambda e: (clamp(e), 0, 0)),
            pl.BlockSpec((1, EXP_H, D_MODEL), lambda e: (clamp(e), 0, 0)),
            pl.BlockSpec((1, 1, D_MODEL), lambda e: (clamp(e), 0, 0)),
        ],
        out_specs=pl.BlockSpec((CAPACITY, D_HALF), lambda e: (e, 0)),
        out_shape=jax.ShapeDtypeStruct((FLAT_ROWS, D_HALF), jnp.float32),
    )(disp, w1, b1.reshape(N_EXPERTS, 1, EXP_H), w2,
      b2.reshape(N_EXPERTS, 1, D_MODEL))


def _worker_id():
    return lax.axis_index("s") * NC + lax.axis_index("c")


@functools.cache
def _sc_kernels():
    mesh = plsc.VectorSubcoreMesh(core_axis_name="c", subcore_axis_name="s",
                                  num_cores=NC, num_subcores=NS)
    scratch = [
        pltpu.VMEM((N_CHUNKS, K_CHUNK), jnp.int32),
        pltpu.VMEM((K_CHUNK, D_HALF), jnp.float32),
        pltpu.VMEM((K_CHUNK, D_HALF), jnp.float32),
        pltpu.SemaphoreType.DMA,
        pltpu.SemaphoreType.DMA,
    ]

    @functools.partial(
        pl.kernel,
        mesh=mesh,
        out_type=jax.ShapeDtypeStruct((FLAT_ROWS, D_HALF), jnp.float32),
        scratch_types=scratch,
    )
    def dispatch(x_hbm, slot_hbm, out_hbm, idx_v, rows0, rows1, sem0, sem1):
        wid = _worker_id()
        rows, sems = (rows0, rows1), (sem0, sem1)
        pltpu.sync_copy(slot_hbm.at[wid], idx_v)
        base = wid * TOK_PER_W
        pltpu.sync_copy(x_hbm.at[pl.ds(base, K_CHUNK)], rows0)
        handles = [None, None]
        for j in range(N_CHUNKS):
            b = j % 2
            handles[b] = pltpu.async_copy(rows[b], out_hbm.at[idx_v.at[j]],
                                          sems[b])
            if j + 1 < N_CHUNKS:
                nb = (j + 1) % 2
                if handles[nb] is not None:
                    handles[nb].wait()
                pltpu.sync_copy(
                    x_hbm.at[pl.ds(base + (j + 1) * K_CHUNK, K_CHUNK)],
                    rows[nb])
        handles[0].wait()
        handles[1].wait()

    @functools.partial(
        pl.kernel,
        mesh=mesh,
        out_type=jax.ShapeDtypeStruct((N_TOKENS, D_HALF), jnp.float32),
        scratch_types=scratch,
    )
    def combine(flat_hbm, slot_hbm, out_hbm, idx_v, rows0, rows1, sem0, sem1):
        wid = _worker_id()
        rows, sems = (rows0, rows1), (sem0, sem1)
        pltpu.sync_copy(slot_hbm.at[wid], idx_v)
        base = wid * TOK_PER_W
        handles = [
            pltpu.async_copy(flat_hbm.at[idx_v.at[0]], rows0, sem0), None]
        for j in range(N_CHUNKS):
            b = j % 2
            if j + 1 < N_CHUNKS:
                handles[1 - b] = pltpu.async_copy(
                    flat_hbm.at[idx_v.at[j + 1]], rows[1 - b], sems[1 - b])
            handles[b].wait()
            pltpu.sync_copy(rows[b],
                            out_hbm.at[pl.ds(base + j * K_CHUNK, K_CHUNK)])

    return dispatch, combine


def _unpack_body(p_ref, out_ref):
    out_ref[...] = _unpack_bf16(p_ref[...]).astype(jnp.float32)


def _final_unpack(packed):
    return pl.pallas_call(
        _unpack_body,
        grid=(N_TBLOCKS,),
        in_specs=[pl.BlockSpec((TB, D_HALF), lambda i: (i, 0))],
        out_specs=pl.BlockSpec((TB, D_MODEL), lambda i: (i, 0)),
        out_shape=jax.ShapeDtypeStruct((N_TOKENS, D_MODEL), jnp.float32),
    )(packed)


def kernel(inputs, Wg1, bg1, Wg2, bg2, W1, b1, W2, b2):
    slot, xbf = _gate_route(inputs, Wg1, bg1.reshape(1, -1),
                            Wg2, bg2.reshape(1, -1))
    slot3 = slot.reshape(NW, N_CHUNKS, K_CHUNK)
    dispatch, combine = _sc_kernels()
    disp = dispatch(xbf, slot3)
    flat = _experts(disp, W1, b1, W2, b2)
    return _final_unpack(combine(flat, slot3))


# speedup vs baseline: 1.0735x; 1.0016x over previous
import functools

import jax
import jax.numpy as jnp
from jax import lax
from jax.experimental import pallas as pl
from jax.experimental.pallas import tpu as pltpu
from jax.experimental.pallas import tpu_sc as plsc

N_TOKENS = 32768
D_MODEL = 768
N_EXPERTS = 64
CAPACITY = 512
GATE_H = 64
EXP_H = 128
FLAT_ROWS = N_EXPERTS * CAPACITY + 1

TB = 1024
N_TBLOCKS = N_TOKENS // TB
D_HALF = D_MODEL // 2


def _pack_bf16(x32):
    lo = lax.bitcast_convert_type(x32[:, :D_HALF].astype(jnp.bfloat16),
                                  jnp.uint16).astype(jnp.uint32)
    hi = lax.bitcast_convert_type(x32[:, D_HALF:].astype(jnp.bfloat16),
                                  jnp.uint16).astype(jnp.uint32)
    return lax.bitcast_convert_type(lo | (hi << 16), jnp.float32)


def _unpack_bf16(p32):
    u = lax.bitcast_convert_type(p32, jnp.uint32)
    lo = lax.bitcast_convert_type((u & 0xFFFF).astype(jnp.uint16),
                                  jnp.bfloat16)
    hi = lax.bitcast_convert_type((u >> 16).astype(jnp.uint16), jnp.bfloat16)
    return jnp.concatenate([lo, hi], axis=1)

NC, NS = 2, 16
NW = NC * NS
TOK_PER_W = N_TOKENS // NW
K_CHUNK = 64
N_CHUNKS = TOK_PER_W // K_CHUNK
N_BUF = 4
N_ROUNDS = N_CHUNKS // N_BUF


def _gate_body(x_ref, wg1_ref, bg1_ref, wg2_ref, bg2_ref, slot_ref, xbf_ref,
               counts_ref):
    i = pl.program_id(0)

    @pl.when(i == 0)
    def _():
        counts_ref[...] = jnp.zeros_like(counts_ref)

    x = x_ref[...]
    xbf_ref[...] = _pack_bf16(x)
    gh = jnp.dot(x, wg1_ref[...], preferred_element_type=jnp.float32)
    gh = jnp.maximum(gh + bg1_ref[...], 0.0)
    logits = jnp.dot(gh, wg2_ref[...], preferred_element_type=jnp.float32)
    logits = logits + bg2_ref[...]

    m = jnp.max(logits, axis=1, keepdims=True)
    iota_e = lax.broadcasted_iota(jnp.int32, (TB, N_EXPERTS), 1)
    sel = logits == m
    expert = jnp.min(jnp.where(sel, iota_e, N_EXPERTS), axis=1)
    first = expert[:, None] == iota_e

    ii = lax.broadcasted_iota(jnp.int32, (TB, TB), 0)
    jj = lax.broadcasted_iota(jnp.int32, (TB, TB), 1)
    tri = (jj <= ii).astype(jnp.bfloat16)
    onehot = first.astype(jnp.bfloat16)
    pos_incl = jnp.dot(tri, onehot, preferred_element_type=jnp.float32)

    prev = counts_ref[...]
    pos_all = pos_incl + prev
    pos_tok = jnp.sum(jnp.where(first, pos_all, 0.0), axis=1) - 1.0
    keep = pos_tok < CAPACITY
    slot = jnp.where(
        keep,
        expert * CAPACITY + pos_tok.astype(jnp.int32),
        N_EXPERTS * CAPACITY,
    )
    slot_ref[...] = slot[None, None, :]
    counts_ref[...] = prev + jnp.sum(first.astype(jnp.float32), axis=0,
                                     keepdims=True)


def _gate_route(x, wg1, bg1, wg2, bg2):
    return pl.pallas_call(
        _gate_body,
        grid=(N_TBLOCKS,),
        in_specs=[
            pl.BlockSpec((TB, D_MODEL), lambda i: (i, 0)),
            pl.BlockSpec((D_MODEL, GATE_H), lambda i: (0, 0)),
            pl.BlockSpec((1, GATE_H), lambda i: (0, 0)),
            pl.BlockSpec((GATE_H, N_EXPERTS), lambda i: (0, 0)),
            pl.BlockSpec((1, N_EXPERTS), lambda i: (0, 0)),
        ],
        out_specs=[
            pl.BlockSpec((1, 1, TB), lambda i: (i, 0, 0)),
            pl.BlockSpec((TB, D_HALF), lambda i: (i, 0)),
        ],
        out_shape=[
            jax.ShapeDtypeStruct((N_TBLOCKS, 1, TB), jnp.int32),
            jax.ShapeDtypeStruct((N_TOKENS, D_HALF), jnp.float32),
        ],
        scratch_shapes=[pltpu.VMEM((1, N_EXPERTS), jnp.float32)],
    )(x, wg1, bg1, wg2, bg2)


def _expert_body(disp_ref, w1_ref, b1_ref, w2_ref, b2_ref, out_ref):
    e = pl.program_id(0)

    @pl.when(e < N_EXPERTS)
    def _():
        xb = _unpack_bf16(disp_ref[...])
        h = jnp.dot(xb, w1_ref[0].astype(jnp.bfloat16),
                    preferred_element_type=jnp.float32)
        h = jnp.maximum(h + b1_ref[0], 0.0)
        z = jnp.dot(h.astype(jnp.bfloat16), w2_ref[0].astype(jnp.bfloat16),
                    preferred_element_type=jnp.float32)
        z = z + b2_ref[0]
        out_ref[...] = _pack_bf16(jax.nn.softmax(z, axis=-1))

    @pl.when(e == N_EXPERTS)
    def _():
        out_ref[...] = jnp.zeros_like(out_ref)


def _experts(disp, w1, b1, w2, b2):
    clamp = lambda e: jnp.minimum(e, N_EXPERTS - 1)
    return pl.pallas_call(
        _expert_body,
        grid=(N_EXPERTS + 1,),
        in_specs=[
            pl.BlockSpec((CAPACITY, D_HALF), lambda e: (clamp(e), 0)),
            pl.BlockSpec((1, D_MODEL, EXP_H), lambda e: (clamp(e), 0, 0)),
            pl.BlockSpec((1, 1, EXP_H), lambda e: (clamp(e), 0, 0)),
            pl.BlockSpec((1, EXP_H, D_MODEL), lambda e: (clamp(e), 0, 0)),
            pl.BlockSpec((1, 1, D_MODEL), lambda e: (clamp(e), 0, 0)),
        ],
        out_specs=pl.BlockSpec((CAPACITY, D_HALF), lambda e: (e, 0)),
        out_shape=jax.ShapeDtypeStruct((FLAT_ROWS, D_HALF), jnp.float32),
    )(disp, w1, b1.reshape(N_EXPERTS, 1, EXP_H), w2,
      b2.reshape(N_EXPERTS, 1, D_MODEL))


def _worker_id():
    return lax.axis_index("s") * NC + lax.axis_index("c")


@functools.cache
def _sc_kernels():
    mesh = plsc.VectorSubcoreMesh(core_axis_name="c", subcore_axis_name="s",
                                  num_cores=NC, num_subcores=NS)
    scratch = (
        [pltpu.VMEM((N_CHUNKS, K_CHUNK), jnp.int32)]
        + [pltpu.VMEM((K_CHUNK, D_HALF), jnp.float32) for _ in range(N_BUF)]
        + [pltpu.SemaphoreType.DMA for _ in range(N_BUF)]
    )

    @functools.partial(
        pl.kernel,
        mesh=mesh,
        out_type=jax.ShapeDtypeStruct((FLAT_ROWS, D_HALF), jnp.float32),
        scratch_types=scratch,
    )
    def dispatch(x_hbm, slot_hbm, out_hbm, idx_v, *bufs):
        rows, sems = bufs[:N_BUF], bufs[N_BUF:]
        wid = _worker_id()
        pltpu.sync_copy(slot_hbm.at[wid], idx_v)
        base = wid * TOK_PER_W
        for g in range(N_ROUNDS):
            for b in range(N_BUF):
                j = g * N_BUF + b
                pltpu.sync_copy(
                    x_hbm.at[pl.ds(base + j * K_CHUNK, K_CHUNK)], rows[b])
            hs = [pltpu.async_copy(rows[b], out_hbm.at[idx_v.at[g * N_BUF + b]],
                                   sems[b]) for b in range(N_BUF)]
            for h in hs:
                h.wait()

    @functools.partial(
        pl.kernel,
        mesh=mesh,
        out_type=jax.ShapeDtypeStruct((N_TOKENS, D_HALF), jnp.float32),
        scratch_types=scratch,
    )
    def combine(flat_hbm, slot_hbm, out_hbm, idx_v, *bufs):
        rows, sems = bufs[:N_BUF], bufs[N_BUF:]
        wid = _worker_id()
        pltpu.sync_copy(slot_hbm.at[wid], idx_v)
        base = wid * TOK_PER_W
        for g in range(N_ROUNDS):
            hs = [pltpu.async_copy(flat_hbm.at[idx_v.at[g * N_BUF + b]],
                                   rows[b], sems[b]) for b in range(N_BUF)]
            for b in range(N_BUF):
                hs[b].wait()
                pltpu.sync_copy(
                    rows[b],
                    out_hbm.at[pl.ds(base + (g * N_BUF + b) * K_CHUNK,
                                     K_CHUNK)])

    return dispatch, combine


def _unpack_body(p_ref, out_ref):
    out_ref[...] = _unpack_bf16(p_ref[...]).astype(jnp.float32)


def _final_unpack(packed):
    return pl.pallas_call(
        _unpack_body,
        grid=(N_TBLOCKS,),
        in_specs=[pl.BlockSpec((TB, D_HALF), lambda i: (i, 0))],
        out_specs=pl.BlockSpec((TB, D_MODEL), lambda i: (i, 0)),
        out_shape=jax.ShapeDtypeStruct((N_TOKENS, D_MODEL), jnp.float32),
    )(packed)


def kernel(inputs, Wg1, bg1, Wg2, bg2, W1, b1, W2, b2):
    slot, xbf = _gate_route(inputs, Wg1, bg1.reshape(1, -1),
                            Wg2, bg2.reshape(1, -1))
    slot3 = slot.reshape(NW, N_CHUNKS, K_CHUNK)
    dispatch, combine = _sc_kernels()
    disp = dispatch(xbf, slot3)
    flat = _experts(disp, W1, b1, W2, b2)
    return _final_unpack(combine(flat, slot3))
